# baseline (speedup 1.0000x reference)
import jax
import jax.numpy as jnp
from jax.experimental import pallas as pl

NMS_THRESH = 0.7
PRE_NMS_TOPK = 2000
POST_NMS_TOPK = 1000
IMG_H = 800.0
IMG_W = 1333.0
B = 256
NT = 8
KPAD = NT * B


def _nms_kernel(x0c_ref, y0c_ref, x1c_ref, y1c_ref, x0r_ref, y0r_ref,
                x1r_ref, y1r_ref, lvlc_ref, lvlr_ref, sc_ref,
                out_ref):
    x0c, y0c, x1c, y1c = x0c_ref[...], y0c_ref[...], x1c_ref[...], y1c_ref[...]
    x0r, y0r, x1r, y1r = x0r_ref[...], y0r_ref[...], x1r_ref[...], y1r_ref[...]
    lvlc, lvlr = lvlc_ref[...], lvlr_ref[...]
    mc = jnp.maximum(jnp.maximum(jnp.max(x0c), jnp.max(y0c)),
                     jnp.maximum(jnp.max(x1c), jnp.max(y1c))) + 1.0

    ox0c = x0c + lvlc * mc
    oy0c = y0c + lvlc * mc
    ox1c = x1c + lvlc * mc
    oy1c = y1c + lvlc * mc
    ox0r = x0r + lvlr * mc
    oy0r = y0r + lvlr * mc
    ox1r = x1r + lvlr * mc
    oy1r = y1r + lvlr * mc

    def over_mat(tj, ti):
        s = slice(tj * B, (tj + 1) * B)
        ax0 = ox0c[s, :]
        ay0 = oy0c[s, :]
        ax1 = ox1c[s, :]
        ay1 = oy1c[s, :]
        bx0 = ox0r[ti:ti + 1, :]
        by0 = oy0r[ti:ti + 1, :]
        bx1 = ox1r[ti:ti + 1, :]
        by1 = oy1r[ti:ti + 1, :]
        area_a = (ax1 - ax0) * (ay1 - ay0)
        area_b = (bx1 - bx0) * (by1 - by0)
        wx = jnp.clip(jnp.minimum(ax1, bx1) - jnp.maximum(ax0, bx0), 0.0)
        wy = jnp.clip(jnp.minimum(ay1, by1) - jnp.maximum(ay0, by0), 0.0)
        inter = wx * wy
        iou = inter / (area_a + area_b - inter + 1e-9)
        return (iou > NMS_THRESH).astype(jnp.float32)

    row_ids = jax.lax.broadcasted_iota(jnp.int32, (B, B), 0)
    col_ids = jax.lax.broadcasted_iota(jnp.int32, (B, B), 1)
    upper = (row_ids < col_ids).astype(jnp.float32)

    keep_rows = []
    for ti in range(NT):
        cnt = jnp.zeros((1, B), dtype=jnp.float32)
        for tj in range(ti):
            cnt = cnt + jnp.dot(keep_rows[tj], over_mat(tj, ti),
                                preferred_element_type=jnp.float32)
        init = (cnt < 0.5).astype(jnp.float32)
        m = over_mat(ti, ti) * upper

        def body(carry):
            k, _ = carry
            s = jnp.dot(k, m, preferred_element_type=jnp.float32)
            k2 = init * (s < 0.5).astype(jnp.float32)
            return k2, jnp.any(k2 != k)

        def cond(carry):
            return carry[1]

        k_fin, _ = jax.lax.while_loop(cond, body, (init, True))
        keep_rows.append(k_fin)

    lane = jax.lax.broadcasted_iota(jnp.int32, (1, B), 1).astype(jnp.float32)
    NOUT = 1024
    out_iota = jax.lax.broadcasted_iota(jnp.int32, (NOUT, B), 0).astype(jnp.float32)

    keptv = []
    supv = []
    cnt_k = []
    cnt_s = []
    for t in range(NT):
        if t < NT - 1:
            valid = jnp.ones((1, B), jnp.float32)
        else:
            valid = (lane < float(PRE_NMS_TOPK - (NT - 1) * B)).astype(
                jnp.float32)
        kv = keep_rows[t] * valid
        sv = (1.0 - keep_rows[t]) * valid
        keptv.append(kv)
        supv.append(sv)
        cnt_k.append(jnp.sum(kv))
        cnt_s.append(jnp.sum(sv))
    nkept = sum(cnt_k)

    out = jnp.zeros((NOUT, 8), jnp.float32)
    base_k = 0.0
    base_s = 0.0
    zcol3 = jnp.zeros((B, 3), jnp.float32)
    negcol = jnp.full((B, 1), -1e10, jnp.float32)
    for t in range(NT):
        s = slice(t * B, (t + 1) * B)
        pay = [x0c[s, :], y0c[s, :], x1c[s, :], y1c[s, :]]
        pay_k = jnp.concatenate(pay + [sc_ref[s, :], zcol3], axis=1)
        pay_s = jnp.concatenate(pay + [negcol, zcol3], axis=1)
        pref_k = jnp.dot(keptv[t], upper, preferred_element_type=jnp.float32)
        pref_s = jnp.dot(supv[t], upper, preferred_element_type=jnp.float32)
        rank_k = jnp.where(keptv[t] > 0.5, base_k + pref_k, -1.0)
        rank_s = jnp.where(supv[t] > 0.5, nkept + base_s + pref_s, -1.0)
        p_k = (out_iota == rank_k).astype(jnp.float32)
        p_s = (out_iota == rank_s).astype(jnp.float32)
        out = out + jnp.dot(p_k, pay_k, preferred_element_type=jnp.float32,
                            precision=jax.lax.Precision.HIGHEST)
        out = out + jnp.dot(p_s, pay_s, preferred_element_type=jnp.float32,
                            precision=jax.lax.Precision.HIGHEST)
        base_k = base_k + cnt_k[t]
        base_s = base_s + cnt_s[t]
    out_ref[...] = out


def kernel(boxes, scores, level_ids):
    top_scores, top_idx = jax.lax.top_k(scores, PRE_NMS_TOPK)
    b = boxes[top_idx]
    lvl = level_ids[top_idx].astype(jnp.float32)
    x0 = jnp.clip(b[:, 0], 0.0, IMG_W)
    y0 = jnp.clip(b[:, 1], 0.0, IMG_H)
    x1 = jnp.clip(b[:, 2], 0.0, IMG_W)
    y1 = jnp.clip(b[:, 3], 0.0, IMG_H)
    bcl = jnp.stack([x0, y0, x1, y1], axis=1)

    pad = KPAD - PRE_NMS_TOPK
    bp = jnp.pad(bcl, ((0, pad), (0, 0)))
    lvlp = jnp.pad(lvl, (0, pad))
    sp = jnp.pad(top_scores, (0, pad))
    cols = [bp[:, i:i + 1] for i in range(4)]
    rows = [bp[:, i].reshape(NT, B) for i in range(4)]
    lvlc = lvlp.reshape(KPAD, 1)
    lvlr = lvlp.reshape(NT, B)
    scol = sp.reshape(KPAD, 1)

    out = pl.pallas_call(
        _nms_kernel,
        out_shape=jax.ShapeDtypeStruct((1024, 8), jnp.float32),
    )(*cols, *rows, lvlc, lvlr, scol)

    return out[:POST_NMS_TOPK, :5]


# speedup vs baseline: 1.3351x; 1.3351x over previous
import jax
import jax.numpy as jnp
from jax.experimental import pallas as pl

NMS_THRESH = 0.7
PRE_NMS_TOPK = 2000
POST_NMS_TOPK = 1000
IMG_H = 800.0
IMG_W = 1333.0
B = 256
NT = 8
KPAD = NT * B


def _nms_kernel(x0r_ref, y0r_ref, x1r_ref, y1r_ref, lvlr_ref,
                x0t_ref, y0t_ref, x1t_ref, y1t_ref, lvlt_ref, keep_ref):
    x0r, y0r, x1r, y1r = x0r_ref[...], y0r_ref[...], x1r_ref[...], y1r_ref[...]
    x0t, y0t, x1t, y1t = x0t_ref[...], y0t_ref[...], x1t_ref[...], y1t_ref[...]
    lvlr, lvlt = lvlr_ref[...], lvlt_ref[...]
    mc = jnp.maximum(jnp.maximum(jnp.max(x0r), jnp.max(y0r)),
                     jnp.maximum(jnp.max(x1r), jnp.max(y1r))) + 1.0

    ox0r = x0r + lvlr * mc
    oy0r = y0r + lvlr * mc
    ox1r = x1r + lvlr * mc
    oy1r = y1r + lvlr * mc
    ox0t = x0t + lvlt * mc
    oy0t = y0t + lvlt * mc
    ox1t = x1t + lvlt * mc
    oy1t = y1t + lvlt * mc
    area_r = (ox1r - ox0r) * (oy1r - oy0r)
    area_t = (ox1t - ox0t) * (oy1t - oy0t)

    def over_mat(tj, ti):
        c = slice(tj, tj + 1)
        ax0, ay0 = ox0t[:, c], oy0t[:, c]
        ax1, ay1 = ox1t[:, c], oy1t[:, c]
        r = slice(ti, ti + 1)
        bx0, by0 = ox0r[r, :], oy0r[r, :]
        bx1, by1 = ox1r[r, :], oy1r[r, :]
        wx = jnp.clip(jnp.minimum(ax1, bx1) - jnp.maximum(ax0, bx0), 0.0)
        wy = jnp.clip(jnp.minimum(ay1, by1) - jnp.maximum(ay0, by0), 0.0)
        inter = wx * wy
        iou = inter / (area_t[:, c] + area_r[r, :] - inter + 1e-9)
        return (iou > NMS_THRESH).astype(jnp.float32)

    row_ids = jax.lax.broadcasted_iota(jnp.int32, (B, B), 0)
    col_ids = jax.lax.broadcasted_iota(jnp.int32, (B, B), 1)
    upper = (row_ids < col_ids).astype(jnp.float32)

    keep_rows = []
    for ti in range(NT):
        cnt = jnp.zeros((1, B), dtype=jnp.float32)
        for tj in range(ti):
            cnt = cnt + jnp.dot(keep_rows[tj], over_mat(tj, ti),
                                preferred_element_type=jnp.float32)
        init = (cnt < 0.5).astype(jnp.float32)
        m = over_mat(ti, ti) * upper

        def body(carry):
            k, _ = carry
            s = jnp.dot(k, m, preferred_element_type=jnp.float32)
            k2 = init * (s < 0.5).astype(jnp.float32)
            return k2, jnp.any(k2 != k)

        def cond(carry):
            return carry[1]

        k_fin, _ = jax.lax.while_loop(cond, body, (init, True))
        keep_rows.append(k_fin)
        keep_ref[ti:ti + 1, :] = k_fin


def kernel(boxes, scores, level_ids):
    top_scores, top_idx = jax.lax.top_k(scores, PRE_NMS_TOPK)
    b = boxes[top_idx]
    lvl = level_ids[top_idx].astype(jnp.float32)
    x0 = jnp.clip(b[:, 0], 0.0, IMG_W)
    y0 = jnp.clip(b[:, 1], 0.0, IMG_H)
    x1 = jnp.clip(b[:, 2], 0.0, IMG_W)
    y1 = jnp.clip(b[:, 3], 0.0, IMG_H)
    bcl = jnp.stack([x0, y0, x1, y1], axis=1)

    pad = KPAD - PRE_NMS_TOPK
    bp = jnp.pad(bcl, ((0, pad), (0, 0)))
    lvlp = jnp.pad(lvl, (0, pad))
    rows = [bp[:, i].reshape(NT, B) for i in range(4)]
    lvlr = lvlp.reshape(NT, B)
    rowts = [r.T for r in rows]
    lvlt = lvlr.T

    keep = pl.pallas_call(
        _nms_kernel,
        out_shape=jax.ShapeDtypeStruct((NT, B), jnp.float32),
    )(*rows, lvlr, *rowts, lvlt)

    keepf = keep.reshape(KPAD)[:PRE_NMS_TOPK]
    keep_scores = jnp.where(keepf > 0.5, top_scores, -1e10)
    final_scores, keep_idx = jax.lax.top_k(keep_scores, POST_NMS_TOPK)
    final_boxes = bcl[keep_idx]
    return jnp.concatenate([final_boxes, final_scores[:, None]], axis=1)
